# baseline (speedup 1.0000x reference)
import functools

import jax
import jax.numpy as jnp
from jax import lax
from jax.experimental import pallas as pl
from jax.experimental.pallas import tpu as pltpu
from jax.experimental.pallas import tpu_sc as plsc

BATCH = 16384
FIELDS = 26
EMBEDDING_DIM = 64

NUM_CORES = 2
NUM_SUBCORES = 16
NW = NUM_CORES * NUM_SUBCORES

B_TOTAL = BATCH * FIELDS
BLK = 128
BATCH_PER_W = BATCH // NW
ROWS_PER_W = BATCH_PER_W * FIELDS
BLOCKS_PER_W = ROWS_PER_W // BLK
NBUF = 4

_mesh = plsc.VectorSubcoreMesh(
    core_axis_name="c", subcore_axis_name="s",
    num_cores=NUM_CORES, num_subcores=NUM_SUBCORES)


@functools.partial(
    pl.kernel,
    out_type=jax.ShapeDtypeStruct((B_TOTAL, EMBEDDING_DIM), jnp.float32),
    mesh=_mesh,
    scratch_types=[
        pltpu.VMEM((FIELDS, BATCH_PER_W), jnp.int32),
        pltpu.VMEM((BLOCKS_PER_W, BLK), jnp.int32),
        pltpu.VMEM((NBUF, BLK, EMBEDDING_DIM), jnp.float32),
        [pltpu.SemaphoreType.DMA] * NBUF,
    ],
    compiler_params=pltpu.CompilerParams(use_tc_tiling_on_sc=False,
                                         needs_layout_passes=False),
)
def _gather_kernel(idxt_hbm, table_hbm, out_hbm, slab_v, idx_v, rows_v, sems):
    wid = lax.axis_index("s") * NUM_CORES + lax.axis_index("c")
    row_base = wid * ROWS_PER_W
    pltpu.sync_copy(idxt_hbm.at[:, pl.ds(wid * BATCH_PER_W, BATCH_PER_W)],
                    slab_v)

    lanes = lax.iota(jnp.int32, 16)

    @pl.loop(0, BLOCKS_PER_W)
    def _reorder(g):
        for k in range(BLK // 16):
            n = g * BLK + k * 16 + lanes
            b = (n * 20165) >> 19
            f = n - b * 26
            vals = plsc.load_gather(slab_v, [f, b])
            idx_v[g, pl.ds(k * 16, 16)] = vals

    def start_gather(g, b):
        pltpu.async_copy(table_hbm.at[idx_v.at[g]], rows_v.at[b], sems[b])

    def wait_gather(b):
        pltpu.make_async_copy(table_hbm.at[idx_v.at[0]], rows_v.at[b],
                              sems[b]).wait()

    for b in range(NBUF - 1):
        start_gather(b, b)

    @pl.loop(0, BLOCKS_PER_W // NBUF)
    def _body(j):
        for b in range(NBUF):
            g = j * NBUF + b
            wait_gather(b)
            gnext = g + NBUF - 1

            @pl.when(gnext < BLOCKS_PER_W)
            def _():
                start_gather(gnext, (b + NBUF - 1) % NBUF)

            pltpu.sync_copy(rows_v.at[b],
                            out_hbm.at[pl.ds(row_base + g * BLK, BLK)])


def kernel(x, weight):
    out = _gather_kernel(x.T, weight)
    return out.reshape(BATCH, FIELDS, EMBEDDING_DIM)


# speedup vs baseline: 1.0974x; 1.0974x over previous
import functools

import jax
import jax.numpy as jnp
from jax import lax
from jax.experimental import pallas as pl
from jax.experimental.pallas import tpu as pltpu
from jax.experimental.pallas import tpu_sc as plsc

BATCH = 16384
FIELDS = 26
EMBEDDING_DIM = 64

NUM_CORES = 2
NUM_SUBCORES = 16
NW = NUM_CORES * NUM_SUBCORES

B_TOTAL = BATCH * FIELDS
BLK = 128
BATCH_PER_W = BATCH // NW
CHUNKS = BATCH_PER_W // BLK
NBUF = 4

_mesh = plsc.VectorSubcoreMesh(
    core_axis_name="c", subcore_axis_name="s",
    num_cores=NUM_CORES, num_subcores=NUM_SUBCORES)


@functools.partial(
    pl.kernel,
    out_type=jax.ShapeDtypeStruct((B_TOTAL,), jnp.int32),
    mesh=_mesh,
    scratch_types=[pltpu.VMEM((8, BATCH_PER_W), jnp.int32)],
)
def _decode_kernel(xt_hbm, out_hbm, vm):
    wid = lax.axis_index("s") * NUM_CORES + lax.axis_index("c")
    col = wid * BATCH_PER_W
    for r in range((FIELDS + 7) // 8):
        nrows = min(8, FIELDS - 8 * r)
        pltpu.sync_copy(
            xt_hbm.at[pl.ds(8 * r, nrows), pl.ds(col, BATCH_PER_W)],
            vm.at[pl.ds(0, nrows)])
        for s in range(nrows):
            f = 8 * r + s
            pltpu.sync_copy(
                vm.at[s],
                out_hbm.at[pl.ds(f * BATCH + col, BATCH_PER_W)])


@functools.partial(
    pl.kernel,
    out_type=jax.ShapeDtypeStruct((BATCH, FIELDS * EMBEDDING_DIM),
                                  jnp.float32),
    mesh=_mesh,
    scratch_types=[
        pltpu.VMEM((FIELDS, BATCH_PER_W), jnp.int32),
        pltpu.VMEM((NBUF, BLK, EMBEDDING_DIM), jnp.float32),
        [pltpu.SemaphoreType.DMA] * NBUF,
    ],
    compiler_params=pltpu.CompilerParams(use_tc_tiling_on_sc=False,
                                         needs_layout_passes=False),
)
def _gather_kernel(idx_hbm, table_hbm, out_hbm, idx_v, rows_v, sems):
    wid = lax.axis_index("s") * NUM_CORES + lax.axis_index("c")
    col = wid * BATCH_PER_W
    for f in range(FIELDS):
        pltpu.sync_copy(idx_hbm.at[pl.ds(f * BATCH + col, BATCH_PER_W)],
                        idx_v.at[f])

    def start_gather(g, b):
        f, cc = g // CHUNKS, g % CHUNKS
        pltpu.async_copy(table_hbm.at[idx_v.at[f, pl.ds(cc * BLK, BLK)]],
                         rows_v.at[b], sems[b])

    def wait_gather(b):
        pltpu.make_async_copy(table_hbm.at[idx_v.at[0, pl.ds(0, BLK)]],
                              rows_v.at[b], sems[b]).wait()

    for b in range(NBUF - 1):
        start_gather(b, b)

    @pl.loop(0, FIELDS)
    def _body(f):
        for cc in range(CHUNKS):
            g = f * CHUNKS + cc
            wait_gather(cc)
            gnext = g + NBUF - 1

            @pl.when(gnext < FIELDS * CHUNKS)
            def _():
                start_gather(gnext, (cc + NBUF - 1) % NBUF)

            pltpu.sync_copy(
                rows_v.at[cc],
                out_hbm.at[pl.ds(col + cc * BLK, BLK),
                           pl.ds(f * EMBEDDING_DIM, EMBEDDING_DIM)])


def kernel(x, weight):
    idx1d = _decode_kernel(x.T)
    out = _gather_kernel(idx1d, weight)
    return out.reshape(BATCH, FIELDS, EMBEDDING_DIM)
